# baseline (speedup 1.0000x reference)
import functools
import jax
import jax.numpy as jnp
from jax import lax
from jax.experimental import pallas as pl
from jax.experimental.pallas import tpu as pltpu
from jax.experimental.pallas import tpu_sc as plsc

NUM_USERS = 25000
NUM_ITEMS = 25000
N_NODES = NUM_USERS + NUM_ITEMS
N_PAD = 50048
D = 64
H = 32
E = 800000
B = 16384
N_LAYERS = 3

NTEC = 16
CH = 128
S = 28
NSUPER = 14
DBUF = 4
GAHEAD = DBUF - 2
NCHUNK = S * NSUPER
EPW = NCHUNK * CH
E_PAD = NTEC * EPW
ROWS_PER_TEC = N_PAD // NTEC

_mesh = plsc.VectorSubcoreMesh(core_axis_name="c", subcore_axis_name="s")
_params = pltpu.CompilerParams(use_tc_tiling_on_sc=False,
                               needs_layout_passes=False)


@functools.partial(
    pl.kernel,
    mesh=_mesh,
    out_type=(
        jax.ShapeDtypeStruct((N_PAD, H), jnp.float32),
        jax.ShapeDtypeStruct((N_PAD, H), jnp.float32),
    ),
    scratch_types=[
        pltpu.VMEM((2 * S, CH), jnp.int32),
        pltpu.VMEM((S, CH), jnp.float32),
        pltpu.VMEM((CH, H), jnp.float32),
        pltpu.VMEM((CH, H), jnp.float32),
        pltpu.VMEM((CH, H), jnp.float32),
        pltpu.VMEM((CH, H), jnp.float32),
        pltpu.VMEM_SHARED((N_PAD, H), jnp.float32),
        pltpu.SemaphoreType.DMA,
        pltpu.SemaphoreType.DMA,
        pltpu.SemaphoreType.DMA,
        pltpu.SemaphoreType.DMA,
        pltpu.SemaphoreType.DMA,
        pltpu.SemaphoreType.DMA,
        pltpu.SemaphoreType.DMA,
        pltpu.SemaphoreType.DMA,
    ],
    compiler_params=_params,
)
def _spmm_layer(xL, xR, zeros_h, idx_h, val_h,
                yL, yR, ebuf, vbuf, rows0, rows1, rows2, rows3,
                acc_sh, gsem0, gsem1, gsem2, gsem3,
                ssem0, ssem1, ssem2, ssem3):
    cid = lax.axis_index("c")
    sid = lax.axis_index("s")
    rows = (rows0, rows1, rows2, rows3)
    gsem = (gsem0, gsem1, gsem2, gsem3)
    ssem = (ssem0, ssem1, ssem2, ssem3)

    def body(x_hbm, y_hbm):
        r0 = sid * ROWS_PER_TEC
        pltpu.sync_copy(zeros_h, acc_sh.at[pl.ds(r0, ROWS_PER_TEC)])
        plsc.subcore_barrier()

        def super_chunk(u, _):
            base2 = (sid * NSUPER + u) * (2 * S)
            basev = (sid * NSUPER + u) * S
            pltpu.sync_copy(idx_h.at[pl.ds(base2, 2 * S)], ebuf)
            pltpu.sync_copy(val_h.at[pl.ds(basev, S)], vbuf)
            gh = {}
            sh = {}
            for c in range(min(GAHEAD, S)):
                b = c % DBUF
                gh[b] = pltpu.async_copy(
                    x_hbm.at[ebuf.at[2 * c + 1]], rows[b], gsem[b])
            for j in range(S):
                b = j % DBUF
                g = j + GAHEAD
                if j >= 1:
                    sh[(j - 1) % DBUF].wait()
                if g < S:
                    gb = g % DBUF
                    gh[gb] = pltpu.async_copy(
                        x_hbm.at[ebuf.at[2 * g + 1]], rows[gb], gsem[gb])
                gh[b].wait()
                j16 = jnp.full((16,), j, jnp.int32)

                @plsc.parallel_loop(0, CH, unroll=8)
                def edge(e, b=b, j16=j16):
                    vv = plsc.load_gather(
                        vbuf, [j16, jnp.broadcast_to(e, (16,))])
                    a = rows[b][e, pl.ds(0, 16)]
                    c2 = rows[b][e, pl.ds(16, 16)]
                    rows[b][e, pl.ds(0, 16)] = a * vv
                    rows[b][e, pl.ds(16, 16)] = c2 * vv
                sh[b] = pltpu.async_copy(rows[b], acc_sh.at[ebuf.at[2 * j]],
                                         ssem[b], add=True)
            sh[(S - 1) % DBUF].wait()
            return 0

        lax.fori_loop(0, NSUPER, super_chunk, 0)
        plsc.subcore_barrier()
        pltpu.sync_copy(acc_sh.at[pl.ds(r0, ROWS_PER_TEC)],
                        y_hbm.at[pl.ds(r0, ROWS_PER_TEC)])

    @pl.when(cid == 0)
    def _():
        body(xL, yL)

    @pl.when(cid == 1)
    def _():
        body(xR, yR)


BCH = 128
NW = 2 * NTEC
B_PER_W = B // NW
NBCHUNK = B_PER_W // BCH


@functools.partial(
    pl.kernel,
    mesh=_mesh,
    out_type=jax.ShapeDtypeStruct((B,), jnp.float32),
    scratch_types=[
        pltpu.VMEM((BCH,), jnp.int32),
        pltpu.VMEM((BCH,), jnp.int32),
        pltpu.VMEM((BCH, H), jnp.float32),
        pltpu.VMEM((BCH, H), jnp.float32),
        pltpu.VMEM((BCH, H), jnp.float32),
        pltpu.VMEM((BCH, H), jnp.float32),
        pltpu.VMEM((BCH, H), jnp.float32),
        pltpu.VMEM((BCH,), jnp.float32),
        pltpu.SemaphoreType.DMA,
    ],
    compiler_params=_params,
)
def _gamma_kernel(x0L, x0R, x1L, x1R, x2L, x2R, x3L, x3R, users_h, items_h,
                  out_h, uidx_v, iidx_v, tmp_v, suL_v, suR_v, siL_v, siR_v,
                  g_v, sem):
    cid = lax.axis_index("c")
    sid = lax.axis_index("s")
    wid = sid * 2 + cid
    base = wid * B_PER_W

    def accum(table, idx_v, dest_v, first):
        pltpu.async_copy(table.at[idx_v], tmp_v, sem).wait()

        def row(e, _):
            for half in range(2):
                s = pl.ds(half * 16, 16)
                t = tmp_v[e, s]
                if first:
                    dest_v[e, s] = t
                else:
                    dest_v[e, s] = dest_v[e, s] + t
            return 0

        lax.fori_loop(0, BCH, row, 0)

    def chunk(i, _):
        off = base + i * BCH
        pltpu.sync_copy(users_h.at[pl.ds(off, BCH)], uidx_v)
        pltpu.sync_copy(items_h.at[pl.ds(off, BCH)], iidx_v)
        accum(x0L, uidx_v, suL_v, True)
        accum(x1L, uidx_v, suL_v, False)
        accum(x2L, uidx_v, suL_v, False)
        accum(x3L, uidx_v, suL_v, False)
        accum(x0R, uidx_v, suR_v, True)
        accum(x1R, uidx_v, suR_v, False)
        accum(x2R, uidx_v, suR_v, False)
        accum(x3R, uidx_v, suR_v, False)
        accum(x0L, iidx_v, siL_v, True)
        accum(x1L, iidx_v, siL_v, False)
        accum(x2L, iidx_v, siL_v, False)
        accum(x3L, iidx_v, siL_v, False)
        accum(x0R, iidx_v, siR_v, True)
        accum(x1R, iidx_v, siR_v, False)
        accum(x2R, iidx_v, siR_v, False)
        accum(x3R, iidx_v, siR_v, False)

        def dot(e, _):
            p = (suL_v[e, pl.ds(0, 16)] * siL_v[e, pl.ds(0, 16)]
                 + suL_v[e, pl.ds(16, 16)] * siL_v[e, pl.ds(16, 16)]
                 + suR_v[e, pl.ds(0, 16)] * siR_v[e, pl.ds(0, 16)]
                 + suR_v[e, pl.ds(16, 16)] * siR_v[e, pl.ds(16, 16)])
            s = jnp.sum(p, axis=0) * (1.0 / 16.0)
            plsc.store_scatter(
                g_v,
                [jnp.broadcast_to(e, (16,))],
                jnp.broadcast_to(s, (16,)),
                mask=lax.iota(jnp.int32, 16) == 0,
            )
            return 0

        lax.fori_loop(0, BCH, dot, 0)
        pltpu.sync_copy(g_v, out_h.at[pl.ds(off, BCH)])
        return 0

    lax.fori_loop(0, NBCHUNK, chunk, 0)


def kernel(users, items, user_emb, item_emb, edge_index, edge_vals):
    x0 = jnp.concatenate(
        [user_emb, item_emb, jnp.zeros((N_PAD - N_NODES, D), jnp.float32)],
        axis=0)
    x0L = x0[:, :H]
    x0R = x0[:, H:]

    pad = E_PAD - E
    dst = jnp.concatenate([edge_index[0], jnp.zeros((pad,), jnp.int32)])
    src = jnp.concatenate([edge_index[1], jnp.zeros((pad,), jnp.int32)])
    val = jnp.concatenate([edge_vals, jnp.zeros((pad,), jnp.float32)])
    dst_r = dst.reshape(NTEC, NSUPER, S, 1, CH)
    src_r = src.reshape(NTEC, NSUPER, S, 1, CH)
    idx_h = jnp.concatenate([dst_r, src_r], axis=3).reshape(-1, CH)
    val_h = val.reshape(-1, CH)
    zeros = jnp.zeros((ROWS_PER_TEC, H), jnp.float32)

    xs = [(x0L, x0R)]
    for _ in range(N_LAYERS):
        yL, yR = _spmm_layer(xs[-1][0], xs[-1][1], zeros, idx_h, val_h)
        xs.append((yL, yR))

    items_n = items + NUM_USERS
    gamma = _gamma_kernel(xs[0][0], xs[0][1], xs[1][0], xs[1][1],
                          xs[2][0], xs[2][1], xs[3][0], xs[3][1],
                          users, items_n)
    return gamma


# speedup vs baseline: 1.0566x; 1.0566x over previous
import functools
import jax
import jax.numpy as jnp
from jax import lax
from jax.experimental import pallas as pl
from jax.experimental.pallas import tpu as pltpu
from jax.experimental.pallas import tpu_sc as plsc

NUM_USERS = 25000
NUM_ITEMS = 25000
N_NODES = NUM_USERS + NUM_ITEMS
N_PAD = 50048
D = 64
Q = 16
E = 800000
B = 16384
N_LAYERS = 3

NTEC = 16
CH = 128
S = 28
NSUPER = 14
DBUF = 4
GAHEAD = DBUF - 2
NCHUNK = S * NSUPER
EPW = NCHUNK * CH
E_PAD = NTEC * EPW
ROWS_PER_TEC = N_PAD // NTEC

_mesh = plsc.VectorSubcoreMesh(core_axis_name="c", subcore_axis_name="s")
_params = pltpu.CompilerParams(use_tc_tiling_on_sc=False,
                               needs_layout_passes=False)


@functools.partial(
    pl.kernel,
    mesh=_mesh,
    out_type=tuple(
        jax.ShapeDtypeStruct((N_PAD, Q), jnp.float32) for _ in range(4)),
    scratch_types=[
        pltpu.VMEM((2 * S, CH), jnp.int32),
        pltpu.VMEM((S, CH), jnp.float32),
    ]
    + [pltpu.VMEM((CH, Q), jnp.float32) for _ in range(2 * DBUF)]
    + [
        pltpu.VMEM_SHARED((N_PAD, Q), jnp.float32),
        pltpu.VMEM_SHARED((N_PAD, Q), jnp.float32),
    ]
    + [pltpu.SemaphoreType.DMA for _ in range(4 * DBUF)],
    compiler_params=_params,
)
def _spmm_layer(x0, x1, x2, x3, zeros_h, idx_h, val_h,
                y0, y1, y2, y3, ebuf, vbuf,
                rowsA0, rowsA1, rowsA2, rowsA3,
                rowsB0, rowsB1, rowsB2, rowsB3,
                accA_sh, accB_sh,
                gsA0, gsA1, gsA2, gsA3, gsB0, gsB1, gsB2, gsB3,
                ssA0, ssA1, ssA2, ssA3, ssB0, ssB1, ssB2, ssB3):
    cid = lax.axis_index("c")
    sid = lax.axis_index("s")
    rowsA = (rowsA0, rowsA1, rowsA2, rowsA3)
    rowsB = (rowsB0, rowsB1, rowsB2, rowsB3)
    gsemA = (gsA0, gsA1, gsA2, gsA3)
    gsemB = (gsB0, gsB1, gsB2, gsB3)
    ssemA = (ssA0, ssA1, ssA2, ssA3)
    ssemB = (ssB0, ssB1, ssB2, ssB3)

    def body(xa_hbm, xb_hbm, ya_hbm, yb_hbm):
        r0 = sid * ROWS_PER_TEC
        pltpu.sync_copy(zeros_h, accA_sh.at[pl.ds(r0, ROWS_PER_TEC)])
        pltpu.sync_copy(zeros_h, accB_sh.at[pl.ds(r0, ROWS_PER_TEC)])
        plsc.subcore_barrier()

        def super_chunk(u, _):
            base2 = (sid * NSUPER + u) * (2 * S)
            basev = (sid * NSUPER + u) * S
            pltpu.sync_copy(idx_h.at[pl.ds(base2, 2 * S)], ebuf)
            pltpu.sync_copy(val_h.at[pl.ds(basev, S)], vbuf)
            ghA = {}
            ghB = {}
            shA = {}
            shB = {}
            for c in range(min(GAHEAD, S)):
                b = c % DBUF
                ghA[b] = pltpu.async_copy(
                    xa_hbm.at[ebuf.at[2 * c + 1]], rowsA[b], gsemA[b])
                ghB[b] = pltpu.async_copy(
                    xb_hbm.at[ebuf.at[2 * c + 1]], rowsB[b], gsemB[b])
            for j in range(S):
                b = j % DBUF
                g = j + GAHEAD
                if g < S:
                    gb = g % DBUF
                    ghA[gb] = pltpu.async_copy(
                        xa_hbm.at[ebuf.at[2 * g + 1]], rowsA[gb], gsemA[gb])
                    ghB[gb] = pltpu.async_copy(
                        xb_hbm.at[ebuf.at[2 * g + 1]], rowsB[gb], gsemB[gb])
                ghA[b].wait()
                ghB[b].wait()
                j16 = jnp.full((16,), j, jnp.int32)

                @plsc.parallel_loop(0, CH, unroll=8)
                def edge(e, b=b, j16=j16):
                    vv = plsc.load_gather(
                        vbuf, [j16, jnp.broadcast_to(e, (16,))])
                    rowsA[b][e, pl.ds(0, 16)] = rowsA[b][e, pl.ds(0, 16)] * vv
                    rowsB[b][e, pl.ds(0, 16)] = rowsB[b][e, pl.ds(0, 16)] * vv
                if j >= 1:
                    shA[(j - 1) % DBUF].wait()
                    shB[(j - 1) % DBUF].wait()
                shA[b] = pltpu.async_copy(
                    rowsA[b], accA_sh.at[ebuf.at[2 * j]], ssemA[b], add=True)
                shB[b] = pltpu.async_copy(
                    rowsB[b], accB_sh.at[ebuf.at[2 * j]], ssemB[b], add=True)
            shA[(S - 1) % DBUF].wait()
            shB[(S - 1) % DBUF].wait()
            return 0

        lax.fori_loop(0, NSUPER, super_chunk, 0)
        plsc.subcore_barrier()
        pltpu.sync_copy(accA_sh.at[pl.ds(r0, ROWS_PER_TEC)],
                        ya_hbm.at[pl.ds(r0, ROWS_PER_TEC)])
        pltpu.sync_copy(accB_sh.at[pl.ds(r0, ROWS_PER_TEC)],
                        yb_hbm.at[pl.ds(r0, ROWS_PER_TEC)])

    @pl.when(cid == 0)
    def _():
        body(x0, x1, y0, y1)

    @pl.when(cid == 1)
    def _():
        body(x2, x3, y2, y3)


BCH = 128
NW = 2 * NTEC
B_PER_W = B // NW
NBCHUNK = B_PER_W // BCH


@functools.partial(
    pl.kernel,
    mesh=_mesh,
    out_type=jax.ShapeDtypeStruct((B,), jnp.float32),
    scratch_types=[
        pltpu.VMEM((BCH,), jnp.int32),
        pltpu.VMEM((BCH,), jnp.int32),
        pltpu.VMEM((BCH, Q), jnp.float32),
        pltpu.VMEM((BCH, Q), jnp.float32),
    ]
    + [pltpu.VMEM((BCH, Q), jnp.float32) for _ in range(8)]
    + [
        pltpu.VMEM((BCH,), jnp.float32),
        pltpu.SemaphoreType.DMA,
        pltpu.SemaphoreType.DMA,
    ],
    compiler_params=_params,
)
def _gamma_kernel(x00, x01, x02, x03, x10, x11, x12, x13,
                  x20, x21, x22, x23, x30, x31, x32, x33,
                  users_h, items_h,
                  out_h, uidx_v, iidx_v, tmp0_v, tmp1_v,
                  su0_v, su1_v, su2_v, su3_v, si0_v, si1_v, si2_v, si3_v,
                  g_v, sem0, sem1):
    cid = lax.axis_index("c")
    sid = lax.axis_index("s")
    wid = sid * 2 + cid
    base = wid * B_PER_W
    tabs = ((x00, x01, x02, x03), (x10, x11, x12, x13),
            (x20, x21, x22, x23), (x30, x31, x32, x33))
    su = (su0_v, su1_v, su2_v, su3_v)
    si = (si0_v, si1_v, si2_v, si3_v)
    tmp = (tmp0_v, tmp1_v)
    sem = (sem0, sem1)

    def chunk(i, _):
        off = base + i * BCH
        pltpu.sync_copy(users_h.at[pl.ds(off, BCH)], uidx_v)
        pltpu.sync_copy(items_h.at[pl.ds(off, BCH)], iidx_v)
        ops = []
        for q in range(4):
            for L in range(4):
                ops.append((tabs[L][q], uidx_v, su[q], L == 0))
        for q in range(4):
            for L in range(4):
                ops.append((tabs[L][q], iidx_v, si[q], L == 0))

        h = {0: pltpu.async_copy(ops[0][0].at[ops[0][1]], tmp0_v, sem0)}
        for k, (_, _, dest_v, first) in enumerate(ops):
            if k + 1 < len(ops):
                nt, nidx, _, _ = ops[k + 1]
                h[(k + 1) % 2] = pltpu.async_copy(
                    nt.at[nidx], tmp[(k + 1) % 2], sem[(k + 1) % 2])
            h[k % 2].wait()
            t_v = tmp[k % 2]

            def row(e, _, t_v=t_v, dest_v=dest_v, first=first):
                t = t_v[e, pl.ds(0, 16)]
                if first:
                    dest_v[e, pl.ds(0, 16)] = t
                else:
                    dest_v[e, pl.ds(0, 16)] = dest_v[e, pl.ds(0, 16)] + t
                return 0

            lax.fori_loop(0, BCH, row, 0)

        def dot(e, _):
            p = (su0_v[e, pl.ds(0, 16)] * si0_v[e, pl.ds(0, 16)]
                 + su1_v[e, pl.ds(0, 16)] * si1_v[e, pl.ds(0, 16)]
                 + su2_v[e, pl.ds(0, 16)] * si2_v[e, pl.ds(0, 16)]
                 + su3_v[e, pl.ds(0, 16)] * si3_v[e, pl.ds(0, 16)])
            s = jnp.sum(p, axis=0) * (1.0 / 16.0)
            plsc.store_scatter(
                g_v,
                [jnp.broadcast_to(e, (16,))],
                jnp.broadcast_to(s, (16,)),
                mask=lax.iota(jnp.int32, 16) == 0,
            )
            return 0

        lax.fori_loop(0, BCH, dot, 0)
        pltpu.sync_copy(g_v, out_h.at[pl.ds(off, BCH)])
        return 0

    lax.fori_loop(0, NBCHUNK, chunk, 0)


def kernel(users, items, user_emb, item_emb, edge_index, edge_vals):
    x0 = jnp.concatenate(
        [user_emb, item_emb, jnp.zeros((N_PAD - N_NODES, D), jnp.float32)],
        axis=0)
    xq = tuple(x0[:, q * Q:(q + 1) * Q] for q in range(4))

    pad = E_PAD - E
    dst = jnp.concatenate([edge_index[0], jnp.zeros((pad,), jnp.int32)])
    src = jnp.concatenate([edge_index[1], jnp.zeros((pad,), jnp.int32)])
    val = jnp.concatenate([edge_vals, jnp.zeros((pad,), jnp.float32)])
    dst_r = dst.reshape(NTEC, NSUPER, S, 1, CH)
    src_r = src.reshape(NTEC, NSUPER, S, 1, CH)
    idx_h = jnp.concatenate([dst_r, src_r], axis=3).reshape(-1, CH)
    val_h = val.reshape(-1, CH)
    zeros = jnp.zeros((ROWS_PER_TEC, Q), jnp.float32)

    xs = [xq]
    for _ in range(N_LAYERS):
        ys = _spmm_layer(*xs[-1], zeros, idx_h, val_h)
        xs.append(tuple(ys))

    items_n = items + NUM_USERS
    tables = [q for layer in xs for q in layer]
    gamma = _gamma_kernel(*tables, users, items_n)
    return gamma
